# baseline (speedup 1.0000x reference)
import functools

import jax
import jax.numpy as jnp
from jax import lax
from jax.experimental import pallas as pl
from jax.experimental.pallas import tpu as pltpu
from jax.experimental.pallas import tpu_sc as plsc

_B = 16384
_NW = 32
_BPW = _B // _NW
_CHUNK = 128
_NCH = _BPW // _CHUNK
_TD = 32
_AD = 32
_L = 16
_RB = 2048


def _relayout_body(src_ref, dst_ref):
    x = src_ref[...]
    eye = (jax.lax.broadcasted_iota(jnp.int32, (32, 32), 0)
           == jax.lax.broadcasted_iota(jnp.int32, (32, 32), 1)
           ).astype(jnp.float32)
    ys = [jax.lax.dot_general(x[:, m * 512:(m + 1) * 512], eye,
                              (((0,), (0,)), ((), ())),
                              preferred_element_type=jnp.float32)
          for m in range(4)]
    dst_ref[...] = jnp.concatenate(ys, axis=1)


def _relayout(table_t, n_rows):
    nblk = (n_rows + _RB - 1) // _RB
    return pl.pallas_call(
        _relayout_body,
        grid=(nblk,),
        in_specs=[pl.BlockSpec((32, _RB), lambda i: (0, i))],
        out_specs=pl.BlockSpec((_RB // 4, 128), lambda i: (i, 0)),
        out_shape=jax.ShapeDtypeStruct((nblk * (_RB // 4), 128), jnp.float32),
    )(table_t)


def _sc_gather_body(idx_t_hbm, idx_a_hbm, ttab_hbm, atab_hbm,
                    t_out, a_out, idx_v, hi_v, rows_v, sem):
    wid = lax.axis_index("s") * 2 + lax.axis_index("c")
    base = wid * _BPW
    for idx_hbm, tab_hbm, out_hbm in (
        (idx_t_hbm, ttab_hbm, t_out),
        (idx_a_hbm, atab_hbm, a_out),
    ):
        pltpu.sync_copy(idx_hbm.at[pl.ds(base, _BPW)], idx_v)
        for k in range(_BPW // _L):
            sl = pl.ds(k * _L, _L)
            r = idx_v[sl]
            hi_v[sl] = jax.lax.shift_left(
                jax.lax.shift_right_logical(r, 11), 9) | (r & 511)
        copies = []
        for j in range(_NCH):
            sl = pl.ds(j * _CHUNK, _CHUNK)
            copies.append(pltpu.async_copy(
                tab_hbm.at[hi_v.at[sl]], rows_v.at[sl], sem))
        for c in copies:
            c.wait()
        pltpu.sync_copy(rows_v, out_hbm.at[pl.ds(base, _BPW)])


def _sc_gather(task_idx, action_idx, rm_task, rm_action):
    mesh = plsc.VectorSubcoreMesh(core_axis_name="c", subcore_axis_name="s")
    f = pl.kernel(
        _sc_gather_body,
        mesh=mesh,
        out_type=(
            jax.ShapeDtypeStruct((_B, 128), jnp.float32),
            jax.ShapeDtypeStruct((_B, 128), jnp.float32),
        ),
        scratch_types=[
            pltpu.VMEM((_BPW,), jnp.int32),
            pltpu.VMEM((_BPW,), jnp.int32),
            pltpu.VMEM((_BPW, 128), jnp.float32),
            pltpu.SemaphoreType.DMA,
        ],
        compiler_params=pltpu.CompilerParams(use_tc_tiling_on_sc=False),
    )
    return f(task_idx, action_idx, rm_task, rm_action)


def _select32(rows128, lo):
    acc = jnp.where(lo == 0, rows128[:, 0:32], 0.0)
    acc = acc + jnp.where(lo == 1, rows128[:, 32:64], 0.0)
    acc = acc + jnp.where(lo == 2, rows128[:, 64:96], 0.0)
    acc = acc + jnp.where(lo == 3, rows128[:, 96:128], 0.0)
    return acc


def _tc_fuse_body(t128_ref, a128_ref, ti_ref, ai_ref, s_ref, o_ref,
                  ws_ref, bs_ref, wo_ref, bo_ref, out_ref):
    t = _select32(t128_ref[...],
                  jax.lax.shift_right_logical(ti_ref[...], 9) & 3).T
    nt = jnp.sqrt(jnp.sum(t * t, axis=0, keepdims=True))
    t = t * jnp.where(nt > 1.0, 1.0 / (nt + 1e-7), 1.0)
    a = _select32(a128_ref[...],
                  jax.lax.shift_right_logical(ai_ref[...], 9) & 3).T
    na = jnp.sqrt(jnp.sum(a * a, axis=0, keepdims=True))
    a = a * jnp.where(na > 1.0, 1.0 / (na + 1e-7), 1.0)
    s = jax.lax.dot_general(ws_ref[...], s_ref[...], (((1,), (1,)), ((), ())),
                            preferred_element_type=jnp.float32)
    s = s + bs_ref[...]
    s = s / jnp.maximum(jnp.sqrt(jnp.sum(s * s, axis=0, keepdims=True)), 1e-12)
    o = jnp.dot(wo_ref[...], o_ref[...], preferred_element_type=jnp.float32)
    o = o + bo_ref[...]
    o = o / jnp.maximum(jnp.sqrt(jnp.sum(o * o, axis=0, keepdims=True)), 1e-12)
    out_ref[...] = jnp.concatenate([t, a, s, o], axis=0)


def _tc_fuse(t128, a128, task2, action2, state, orientation_t, W_state_t,
             b_state, W_orient_t, b_orient, block_cols=2048):
    nblk = _B // block_cols
    col_blk = lambda i: (0, i)
    row_blk = lambda i: (i, 0)
    rep = lambda i: (0, 0)
    return pl.pallas_call(
        _tc_fuse_body,
        grid=(nblk,),
        in_specs=[
            pl.BlockSpec((block_cols, 128), row_blk),
            pl.BlockSpec((block_cols, 128), row_blk),
            pl.BlockSpec((block_cols, 1), row_blk),
            pl.BlockSpec((block_cols, 1), row_blk),
            pl.BlockSpec((block_cols, 128), row_blk),
            pl.BlockSpec((16, block_cols), col_blk),
            pl.BlockSpec((64, 128), rep),
            pl.BlockSpec((64, 1), rep),
            pl.BlockSpec((64, 16), rep),
            pl.BlockSpec((64, 1), rep),
        ],
        out_specs=pl.BlockSpec((_TD + _AD + 128, block_cols), col_blk),
        out_shape=jax.ShapeDtypeStruct((_TD + _AD + 128, _B), jnp.float32),
    )(t128, a128, task2, action2, state, orientation_t, W_state_t, b_state,
      W_orient_t, b_orient)


def kernel(task, action, state, orientation, task_table, action_table,
           W_state, b_state, W_orient, b_orient):
    rm_task = _relayout(task_table.T, task_table.shape[0])
    rm_action = _relayout(action_table.T, action_table.shape[0])
    t128, a128 = _sc_gather(task, action, rm_task, rm_action)
    out_t = _tc_fuse(t128, a128, task.reshape(-1, 1), action.reshape(-1, 1),
                     state, orientation.T, W_state.T, b_state.reshape(-1, 1),
                     W_orient.T, b_orient.reshape(-1, 1))
    return out_t.T


# speedup vs baseline: 1.5148x; 1.5148x over previous
import functools

import jax
import jax.numpy as jnp
from jax import lax
from jax.experimental import pallas as pl
from jax.experimental.pallas import tpu as pltpu
from jax.experimental.pallas import tpu_sc as plsc

_B = 16384
_NW = 32
_BPW = _B // _NW
_CHUNK = 128
_NCH = _BPW // _CHUNK
_TD = 32
_AD = 32
_L = 16
_RB = 8192
_SRB = 13
_SM4 = 11


def _relayout_body(src_ref, dst_ref):
    x = src_ref[...]
    eye = (jax.lax.broadcasted_iota(jnp.int32, (32, 32), 0)
           == jax.lax.broadcasted_iota(jnp.int32, (32, 32), 1)
           ).astype(jnp.float32)
    m4 = _RB // 4
    ys = [jax.lax.dot_general(x[:, m * m4:(m + 1) * m4], eye,
                              (((0,), (0,)), ((), ())),
                              preferred_element_type=jnp.float32)
          for m in range(4)]
    dst_ref[...] = jnp.concatenate(ys, axis=1)


def _relayout(table_t, n_rows):
    nblk = (n_rows + _RB - 1) // _RB
    return pl.pallas_call(
        _relayout_body,
        grid=(nblk,),
        in_specs=[pl.BlockSpec((32, _RB), lambda i: (0, i))],
        out_specs=pl.BlockSpec((_RB // 4, 128), lambda i: (i, 0)),
        out_shape=jax.ShapeDtypeStruct((nblk * (_RB // 4), 128), jnp.float32),
    )(table_t)


def _sc_gather_body(idx_t_hbm, idx_a_hbm, ttab_hbm, atab_hbm,
                    t_out, a_out, idx_v, hi_v, rows_v, sem):
    wid = lax.axis_index("s") * 2 + lax.axis_index("c")
    base = wid * _BPW
    for idx_hbm, tab_hbm, out_hbm in (
        (idx_t_hbm, ttab_hbm, t_out),
        (idx_a_hbm, atab_hbm, a_out),
    ):
        pltpu.sync_copy(idx_hbm.at[pl.ds(base, _BPW)], idx_v)
        for k in range(_BPW // _L):
            sl = pl.ds(k * _L, _L)
            r = idx_v[sl]
            hi_v[sl] = jax.lax.shift_left(
                jax.lax.shift_right_logical(r, _SRB), _SM4) | (
                    r & ((_RB // 4) - 1))
        copies = []
        for j in range(_NCH):
            sl = pl.ds(j * _CHUNK, _CHUNK)
            copies.append(pltpu.async_copy(
                tab_hbm.at[hi_v.at[sl]], rows_v.at[sl], sem))
        for c in copies:
            c.wait()
        pltpu.sync_copy(rows_v, out_hbm.at[pl.ds(base, _BPW)])


def _sc_gather(task_idx, action_idx, rm_task, rm_action):
    mesh = plsc.VectorSubcoreMesh(core_axis_name="c", subcore_axis_name="s")
    f = pl.kernel(
        _sc_gather_body,
        mesh=mesh,
        out_type=(
            jax.ShapeDtypeStruct((_B, 128), jnp.float32),
            jax.ShapeDtypeStruct((_B, 128), jnp.float32),
        ),
        scratch_types=[
            pltpu.VMEM((_BPW,), jnp.int32),
            pltpu.VMEM((_BPW,), jnp.int32),
            pltpu.VMEM((_BPW, 128), jnp.float32),
            pltpu.SemaphoreType.DMA,
        ],
        compiler_params=pltpu.CompilerParams(use_tc_tiling_on_sc=False),
    )
    return f(task_idx, action_idx, rm_task, rm_action)


def _select32(rows128, lo):
    acc = jnp.where(lo == 0, rows128[:, 0:32], 0.0)
    acc = acc + jnp.where(lo == 1, rows128[:, 32:64], 0.0)
    acc = acc + jnp.where(lo == 2, rows128[:, 64:96], 0.0)
    acc = acc + jnp.where(lo == 3, rows128[:, 96:128], 0.0)
    return acc


def _tc_fuse_body(t128_ref, a128_ref, ti_ref, ai_ref, s_ref, o_ref,
                  ws_ref, bs_ref, wo_ref, bo_ref, out_ref):
    t = _select32(t128_ref[...],
                  jax.lax.shift_right_logical(ti_ref[...], _SM4) & 3).T
    nt = jnp.sqrt(jnp.sum(t * t, axis=0, keepdims=True))
    t = t * jnp.where(nt > 1.0, 1.0 / (nt + 1e-7), 1.0)
    a = _select32(a128_ref[...],
                  jax.lax.shift_right_logical(ai_ref[...], _SM4) & 3).T
    na = jnp.sqrt(jnp.sum(a * a, axis=0, keepdims=True))
    a = a * jnp.where(na > 1.0, 1.0 / (na + 1e-7), 1.0)
    s = jax.lax.dot_general(ws_ref[...], s_ref[...], (((1,), (1,)), ((), ())),
                            preferred_element_type=jnp.float32)
    s = s + bs_ref[...]
    s = s / jnp.maximum(jnp.sqrt(jnp.sum(s * s, axis=0, keepdims=True)), 1e-12)
    o = jnp.dot(wo_ref[...], o_ref[...], preferred_element_type=jnp.float32)
    o = o + bo_ref[...]
    o = o / jnp.maximum(jnp.sqrt(jnp.sum(o * o, axis=0, keepdims=True)), 1e-12)
    out_ref[...] = jnp.concatenate([t, a, s, o], axis=0)


def _tc_fuse(t128, a128, task2, action2, state, orientation_t, W_state_t,
             b_state, W_orient_t, b_orient, block_cols=2048):
    nblk = _B // block_cols
    col_blk = lambda i: (0, i)
    row_blk = lambda i: (i, 0)
    rep = lambda i: (0, 0)
    return pl.pallas_call(
        _tc_fuse_body,
        grid=(nblk,),
        in_specs=[
            pl.BlockSpec((block_cols, 128), row_blk),
            pl.BlockSpec((block_cols, 128), row_blk),
            pl.BlockSpec((block_cols, 1), row_blk),
            pl.BlockSpec((block_cols, 1), row_blk),
            pl.BlockSpec((block_cols, 128), row_blk),
            pl.BlockSpec((16, block_cols), col_blk),
            pl.BlockSpec((64, 128), rep),
            pl.BlockSpec((64, 1), rep),
            pl.BlockSpec((64, 16), rep),
            pl.BlockSpec((64, 1), rep),
        ],
        out_specs=pl.BlockSpec((_TD + _AD + 128, block_cols), col_blk),
        out_shape=jax.ShapeDtypeStruct((_TD + _AD + 128, _B), jnp.float32),
    )(t128, a128, task2, action2, state, orientation_t, W_state_t, b_state,
      W_orient_t, b_orient)


def kernel(task, action, state, orientation, task_table, action_table,
           W_state, b_state, W_orient, b_orient):
    rm_task = _relayout(task_table.T, task_table.shape[0])
    rm_action = _relayout(action_table.T, action_table.shape[0])
    t128, a128 = _sc_gather(task, action, rm_task, rm_action)
    out_t = _tc_fuse(t128, a128, task.reshape(-1, 1), action.reshape(-1, 1),
                     state, orientation.T, W_state.T, b_state.reshape(-1, 1),
                     W_orient.T, b_orient.reshape(-1, 1))
    return out_t.T


# speedup vs baseline: 1.5155x; 1.0005x over previous
import functools

import jax
import jax.numpy as jnp
from jax import lax
from jax.experimental import pallas as pl
from jax.experimental.pallas import tpu as pltpu
from jax.experimental.pallas import tpu_sc as plsc

_B = 16384
_NW = 32
_BPW = _B // _NW
_CHUNK = 128
_NCH = _BPW // _CHUNK
_TD = 32
_AD = 32
_L = 16
_RB = 32768
_SRB = 15
_SM4 = 13


def _relayout_body(src_ref, dst_ref):
    x = src_ref[...]
    eye = (jax.lax.broadcasted_iota(jnp.int32, (32, 32), 0)
           == jax.lax.broadcasted_iota(jnp.int32, (32, 32), 1)
           ).astype(jnp.float32)
    m4 = _RB // 4
    ys = [jax.lax.dot_general(x[:, m * m4:(m + 1) * m4], eye,
                              (((0,), (0,)), ((), ())),
                              preferred_element_type=jnp.float32)
          for m in range(4)]
    dst_ref[...] = jnp.concatenate(ys, axis=1)


def _relayout(table_t, n_rows):
    nblk = (n_rows + _RB - 1) // _RB
    return pl.pallas_call(
        _relayout_body,
        grid=(nblk,),
        in_specs=[pl.BlockSpec((32, _RB), lambda i: (0, i))],
        out_specs=pl.BlockSpec((_RB // 4, 128), lambda i: (i, 0)),
        out_shape=jax.ShapeDtypeStruct((nblk * (_RB // 4), 128), jnp.float32),
    )(table_t)


def _sc_gather_body(idx_t_hbm, idx_a_hbm, ttab_hbm, atab_hbm,
                    t_out, a_out, idx_v, hi_v, rows_v, sem):
    wid = lax.axis_index("s") * 2 + lax.axis_index("c")
    base = wid * _BPW
    for idx_hbm, tab_hbm, out_hbm in (
        (idx_t_hbm, ttab_hbm, t_out),
        (idx_a_hbm, atab_hbm, a_out),
    ):
        pltpu.sync_copy(idx_hbm.at[pl.ds(base, _BPW)], idx_v)
        for k in range(_BPW // _L):
            sl = pl.ds(k * _L, _L)
            r = idx_v[sl]
            hi_v[sl] = jax.lax.shift_left(
                jax.lax.shift_right_logical(r, _SRB), _SM4) | (
                    r & ((_RB // 4) - 1))
        copies = []
        for j in range(_NCH):
            sl = pl.ds(j * _CHUNK, _CHUNK)
            copies.append(pltpu.async_copy(
                tab_hbm.at[hi_v.at[sl]], rows_v.at[sl], sem))
        for c in copies:
            c.wait()
        pltpu.sync_copy(rows_v, out_hbm.at[pl.ds(base, _BPW)])


def _sc_gather(task_idx, action_idx, rm_task, rm_action):
    mesh = plsc.VectorSubcoreMesh(core_axis_name="c", subcore_axis_name="s")
    f = pl.kernel(
        _sc_gather_body,
        mesh=mesh,
        out_type=(
            jax.ShapeDtypeStruct((_B, 128), jnp.float32),
            jax.ShapeDtypeStruct((_B, 128), jnp.float32),
        ),
        scratch_types=[
            pltpu.VMEM((_BPW,), jnp.int32),
            pltpu.VMEM((_BPW,), jnp.int32),
            pltpu.VMEM((_BPW, 128), jnp.float32),
            pltpu.SemaphoreType.DMA,
        ],
        compiler_params=pltpu.CompilerParams(use_tc_tiling_on_sc=False),
    )
    return f(task_idx, action_idx, rm_task, rm_action)


def _select32(rows128, lo):
    acc = jnp.where(lo == 0, rows128[:, 0:32], 0.0)
    acc = acc + jnp.where(lo == 1, rows128[:, 32:64], 0.0)
    acc = acc + jnp.where(lo == 2, rows128[:, 64:96], 0.0)
    acc = acc + jnp.where(lo == 3, rows128[:, 96:128], 0.0)
    return acc


def _tc_fuse_body(t128_ref, a128_ref, ti_ref, ai_ref, s_ref, o_ref,
                  ws_ref, bs_ref, wo_ref, bo_ref, out_ref):
    t = _select32(t128_ref[...],
                  jax.lax.shift_right_logical(ti_ref[...], _SM4) & 3).T
    nt = jnp.sqrt(jnp.sum(t * t, axis=0, keepdims=True))
    t = t * jnp.where(nt > 1.0, 1.0 / (nt + 1e-7), 1.0)
    a = _select32(a128_ref[...],
                  jax.lax.shift_right_logical(ai_ref[...], _SM4) & 3).T
    na = jnp.sqrt(jnp.sum(a * a, axis=0, keepdims=True))
    a = a * jnp.where(na > 1.0, 1.0 / (na + 1e-7), 1.0)
    s = jax.lax.dot_general(ws_ref[...], s_ref[...], (((1,), (1,)), ((), ())),
                            preferred_element_type=jnp.float32)
    s = s + bs_ref[...]
    s = s / jnp.maximum(jnp.sqrt(jnp.sum(s * s, axis=0, keepdims=True)), 1e-12)
    o = jnp.dot(wo_ref[...], o_ref[...], preferred_element_type=jnp.float32)
    o = o + bo_ref[...]
    o = o / jnp.maximum(jnp.sqrt(jnp.sum(o * o, axis=0, keepdims=True)), 1e-12)
    out_ref[...] = jnp.concatenate([t, a, s, o], axis=0)


def _tc_fuse(t128, a128, task2, action2, state, orientation_t, W_state_t,
             b_state, W_orient_t, b_orient, block_cols=2048):
    nblk = _B // block_cols
    col_blk = lambda i: (0, i)
    row_blk = lambda i: (i, 0)
    rep = lambda i: (0, 0)
    return pl.pallas_call(
        _tc_fuse_body,
        grid=(nblk,),
        in_specs=[
            pl.BlockSpec((block_cols, 128), row_blk),
            pl.BlockSpec((block_cols, 128), row_blk),
            pl.BlockSpec((block_cols, 1), row_blk),
            pl.BlockSpec((block_cols, 1), row_blk),
            pl.BlockSpec((block_cols, 128), row_blk),
            pl.BlockSpec((16, block_cols), col_blk),
            pl.BlockSpec((64, 128), rep),
            pl.BlockSpec((64, 1), rep),
            pl.BlockSpec((64, 16), rep),
            pl.BlockSpec((64, 1), rep),
        ],
        out_specs=pl.BlockSpec((_TD + _AD + 128, block_cols), col_blk),
        out_shape=jax.ShapeDtypeStruct((_TD + _AD + 128, _B), jnp.float32),
    )(t128, a128, task2, action2, state, orientation_t, W_state_t, b_state,
      W_orient_t, b_orient)


def kernel(task, action, state, orientation, task_table, action_table,
           W_state, b_state, W_orient, b_orient):
    rm_task = _relayout(task_table.T, task_table.shape[0])
    rm_action = _relayout(action_table.T, action_table.shape[0])
    t128, a128 = _sc_gather(task, action, rm_task, rm_action)
    out_t = _tc_fuse(t128, a128, task.reshape(-1, 1), action.reshape(-1, 1),
                     state, orientation.T, W_state.T, b_state.reshape(-1, 1),
                     W_orient.T, b_orient.reshape(-1, 1))
    return out_t.T
